# baseline (speedup 1.0000x reference)
import functools

import jax
import jax.numpy as jnp
from jax import lax
from jax.experimental import pallas as pl
from jax.experimental.pallas import tpu as pltpu
from jax.experimental.pallas import tpu_sc as plsc

V = 900200
PAD = 600000
D = 16
B = 16384
NSLOT = 96
MLP_IN = 96
H1, H2 = 256, 128

SEGS = ((0, 1), (1, 2), (2, 3), (3, 23), (23, 73), (73, 93))

NC = 2
NS = 16
NW = NC * NS
PER_W = B // NW
CHUNK = 8
NCHUNK = PER_W // CHUNK
CIDX = CHUNK * NSLOT
NV = NSLOT // D


TBV = 32768
NTB = 28
VP = TBV * NTB


def _tc_transpose_body(in_ref, out_ref):
    x = in_ref[...]
    eye = (lax.broadcasted_iota(jnp.int32, (16, 16), 0)
           == lax.broadcasted_iota(jnp.int32, (16, 16), 1)
           ).astype(jnp.float32)
    for a in range(8):
        piece = lax.dot_general(
            x[:, 4096 * a:4096 * (a + 1)], eye,
            (((0,), (0,)), ((), ())),
            preferred_element_type=jnp.float32)
        out_ref[:, pl.ds(16 * a, 16)] = piece


def _format_table(emb_t):
    out = pl.pallas_call(
        _tc_transpose_body,
        grid=(NTB,),
        in_specs=[pl.BlockSpec((16, TBV), lambda i: (0, i))],
        out_specs=pl.BlockSpec((TBV // 8, 128), lambda i: (i, 0)),
        out_shape=jax.ShapeDtypeStruct((VP // 8, 128), jnp.float32),
    )(emb_t)
    return out.reshape(VP, D)


def _sc_gather_pool(emb_table, fc_flat, idx_flat):
    mesh = plsc.VectorSubcoreMesh(core_axis_name="c", subcore_axis_name="s")

    @functools.partial(
        pl.kernel,
        mesh=mesh,
        compiler_params=pltpu.CompilerParams(use_tc_tiling_on_sc=False,
                                             needs_layout_passes=False),
        out_type=(
            jax.ShapeDtypeStruct((B, NSLOT), jnp.float32),
            jax.ShapeDtypeStruct((B, D), jnp.float32),
        ),
        scratch_types=[
            pltpu.VMEM((CIDX,), jnp.int32),
            pltpu.VMEM((CIDX,), jnp.int32),
            pltpu.VMEM((CIDX, D), jnp.float32),
            pltpu.VMEM((CIDX,), jnp.float32),
            pltpu.VMEM((CHUNK, NSLOT), jnp.float32),
            pltpu.VMEM((CHUNK, D), jnp.float32),
            pltpu.SemaphoreType.DMA,
            pltpu.SemaphoreType.DMA,
        ],
    )
    def k(emb_hbm, fc_hbm, idx_hbm,
          embx_hbm, fcp_hbm,
          idx_v, idxp_v, rows_v, fc_v, out_v, fcp_v, sem_e, sem_f):
        wid = lax.axis_index("s") * NC + lax.axis_index("c")
        lane = lax.broadcasted_iota(jnp.int32, (D,), 0)

        def chunk_body(c, carry):
            s0 = wid * PER_W + c * CHUNK
            pltpu.sync_copy(idx_hbm.at[pl.ds(s0 * NSLOT, CIDX)], idx_v)
            for j in range(CIDX // D):
                sl16 = pl.ds(j * D, D)
                v = idx_v[sl16]
                low = jnp.bitwise_and(v, 32767)
                p = (jnp.bitwise_and(v, ~32767)
                     | jnp.left_shift(jnp.bitwise_and(low, 4095), 3)
                     | jnp.right_shift(low, 12))
                idxp_v[sl16] = p
            cps = []
            for s in range(CHUNK):
                sl = pl.ds(s * NSLOT, NSLOT)
                cps.append(pltpu.async_copy(
                    emb_hbm.at[idxp_v.at[sl]], rows_v.at[sl], sem_e))
                cps.append(pltpu.async_copy(
                    fc_hbm.at[idx_v.at[sl]], fc_v.at[sl], sem_f))
            for cp in cps:
                cp.wait()

            def s_body(s, carry2):
                rb = s * NSLOT
                iv = [idx_v[pl.ds(rb + j * D, D)] for j in range(NV)]
                nz = [v != PAD for v in iv]

                def pcnt(j, lo=None, hi=None):
                    m = nz[j]
                    if lo is not None:
                        m = jnp.logical_and(m, lane >= lo)
                    if hi is not None:
                        m = jnp.logical_and(m, lane < hi)
                    return plsc.all_reduce_population_count(m)

                cg = pcnt(0, lo=3) + pcnt(1, hi=7)
                cw = pcnt(1, lo=7) + pcnt(2) + pcnt(3) + pcnt(4, hi=9)
                cd = pcnt(4, lo=9) + pcnt(5)
                inv_g = 1.0 / (cg.astype(jnp.float32) + 1e-08)
                inv_w = 1.0 / (cw.astype(jnp.float32) + 1e-08)
                inv_d = 1.0 / (cd.astype(jnp.float32) + 1e-08)

                invs = (None, None, None, inv_g, inv_w, inv_d)
                for f, (lo, hi) in enumerate(SEGS):
                    acc = rows_v[rb + lo]
                    for r in range(lo + 1, hi):
                        acc = acc + rows_v[rb + r]
                    if invs[f] is not None:
                        acc = acc * invs[f]
                    out_v[s, pl.ds(f * D, D)] = acc

                one = jnp.ones((D,), jnp.float32)
                scales = (
                    jnp.where(lane < 3, one, inv_g),
                    jnp.where(lane < 7, inv_g, inv_w),
                    inv_w,
                    inv_w,
                    jnp.where(lane < 9, inv_w, inv_d),
                    inv_d,
                )
                part = jnp.zeros((D,), jnp.float32)
                for j in range(NV):
                    part = part + fc_v[pl.ds(rb + j * D, D)] * scales[j]
                fcp_v[s] = part
                return carry2

            lax.fori_loop(0, CHUNK, s_body, 0)
            pltpu.sync_copy(out_v, embx_hbm.at[pl.ds(s0, CHUNK), :])
            pltpu.sync_copy(fcp_v, fcp_hbm.at[pl.ds(s0, CHUNK), :])
            return carry

        lax.fori_loop(0, NCHUNK, chunk_body, 0)

    return k(emb_table, fc_flat, idx_flat)


def _tc_body(embx_ref, fcp_ref, W1_ref, b1_ref, W2_ref, b2_ref,
             W3_ref, b3_ref, bias_ref, out_ref):
    ex = embx_ref[...]
    fm1 = bias_ref[0, 0] + jnp.sum(fcp_ref[...], axis=1, keepdims=True)

    s1 = jnp.sum(ex, axis=1, keepdims=True)
    s2 = jnp.sum(ex * ex, axis=1, keepdims=True)
    fm2 = 0.5 * (s1 * s1 - s2)

    h = jnp.maximum(
        jnp.dot(ex, W1_ref[...], preferred_element_type=jnp.float32)
        + b1_ref[...], 0.0)
    h = jnp.maximum(
        jnp.dot(h, W2_ref[...], preferred_element_type=jnp.float32)
        + b2_ref[...], 0.0)
    mlp = (jnp.dot(h, W3_ref[...], preferred_element_type=jnp.float32)
           + b3_ref[0, 0])

    out_ref[...] = jax.nn.sigmoid(fm1 + fm2 + mlp)


def kernel(user, item, genres, writers, directors, year,
           emb_table, fc_table, bias, W1, b1, W2, b2, W3, b3):
    i32 = jnp.int32
    idx_all = jnp.concatenate([
        user[:, None].astype(i32), item[:, None].astype(i32),
        year[:, None].astype(i32), genres.astype(i32),
        writers.astype(i32), directors.astype(i32),
        jnp.full((B, 3), PAD, dtype=i32),
    ], axis=1)

    emb_lin = _format_table(emb_table.T)
    embx, fcp = _sc_gather_pool(
        emb_lin, fc_table[:, 0], idx_all.reshape(-1))

    W1p = jnp.concatenate([W1[0:32], W1[80:96], W1[32:80]], axis=0)

    bm = 2048
    grid = (B // bm,)
    full = lambda i: (0, 0)
    y = pl.pallas_call(
        _tc_body,
        grid=grid,
        in_specs=[
            pl.BlockSpec((bm, NSLOT), lambda i: (i, 0)),
            pl.BlockSpec((bm, D), lambda i: (i, 0)),
            pl.BlockSpec((MLP_IN, H1), full),
            pl.BlockSpec((1, H1), full),
            pl.BlockSpec((H1, H2), full),
            pl.BlockSpec((1, H2), full),
            pl.BlockSpec((H2, 1), full),
            pl.BlockSpec((1, 1), full),
            pl.BlockSpec((1, 1), full),
        ],
        out_specs=pl.BlockSpec((bm, 1), lambda i: (i, 0)),
        out_shape=jax.ShapeDtypeStruct((B, 1), jnp.float32),
    )(embx, fcp, W1p, b1.reshape(1, H1), W2, b2.reshape(1, H2),
      W3, b3.reshape(1, 1), bias.reshape(1, 1))
    return y[:, 0]


# speedup vs baseline: 1.1405x; 1.1405x over previous
import functools

import jax
import jax.numpy as jnp
from jax import lax
from jax.experimental import pallas as pl
from jax.experimental.pallas import tpu as pltpu
from jax.experimental.pallas import tpu_sc as plsc

V = 900200
PAD = 600000
D = 16
B = 16384
NSLOT = 96
MLP_IN = 96
H1, H2 = 256, 128

SEGS = ((0, 1), (1, 2), (2, 3), (3, 23), (23, 73), (73, 93))

NC = 2
NS = 16
NW = NC * NS
PER_W = B // NW
CHUNK = 8
NCHUNK = PER_W // CHUNK
CIDX = CHUNK * NSLOT
NV = NSLOT // D


TBV = 32768
NTB = 28
VP = TBV * NTB


def _tc_transpose_body(in_ref, out_ref):
    x = in_ref[...]
    def _mask_tail(xv):
        colg = (TBV * (NTB - 1)
                + lax.broadcasted_iota(jnp.int32, (16, TBV), 1))
        return jnp.where(colg < V, xv, 0.0)

    x = lax.cond(pl.program_id(0) == NTB - 1, _mask_tail, lambda xv: xv, x)
    row = lax.broadcasted_iota(jnp.int32, (16, 128), 0)
    col = lax.broadcasted_iota(jnp.int32, (16, 128), 1)
    acc = jnp.zeros((TBV // 8, 128), jnp.float32)
    for a in range(8):
        place = (col == row + 16 * a).astype(jnp.float32)
        acc = acc + lax.dot_general(
            x[:, 4096 * a:4096 * (a + 1)], place,
            (((0,), (0,)), ((), ())),
            preferred_element_type=jnp.float32)
    out_ref[...] = acc


def _format_table(emb_t):
    out = pl.pallas_call(
        _tc_transpose_body,
        grid=(NTB,),
        in_specs=[pl.BlockSpec((16, TBV), lambda i: (0, i))],
        out_specs=pl.BlockSpec((TBV // 8, 128), lambda i: (i, 0)),
        out_shape=jax.ShapeDtypeStruct((VP // 8, 128), jnp.float32),
    )(emb_t)
    return out.reshape(VP, D)


def _sc_gather_pool(emb_table, fc_flat, idx_flat):
    mesh = plsc.VectorSubcoreMesh(core_axis_name="c", subcore_axis_name="s")

    @functools.partial(
        pl.kernel,
        mesh=mesh,
        compiler_params=pltpu.CompilerParams(use_tc_tiling_on_sc=False,
                                             needs_layout_passes=False),
        out_type=(
            jax.ShapeDtypeStruct((B, NSLOT), jnp.float32),
            jax.ShapeDtypeStruct((B, D), jnp.float32),
        ),
        scratch_types=[
            pltpu.VMEM((CIDX,), jnp.int32),
            pltpu.VMEM((CIDX,), jnp.int32),
            pltpu.VMEM((CIDX, D), jnp.float32),
            pltpu.VMEM((CIDX,), jnp.float32),
            pltpu.VMEM((CHUNK, NSLOT), jnp.float32),
            pltpu.VMEM((CHUNK, D), jnp.float32),
            pltpu.SemaphoreType.DMA,
            pltpu.SemaphoreType.DMA,
        ],
    )
    def k(emb_hbm, fc_hbm, idx_hbm,
          embx_hbm, fcp_hbm,
          idx_v, idxp_v, rows_v, fc_v, out_v, fcp_v, sem_e, sem_f):
        wid = lax.axis_index("s") * NC + lax.axis_index("c")
        lane = lax.broadcasted_iota(jnp.int32, (D,), 0)

        def chunk_body(c, carry):
            s0 = wid * PER_W + c * CHUNK
            pltpu.sync_copy(idx_hbm.at[pl.ds(s0 * NSLOT, CIDX)], idx_v)
            for j in range(CIDX // D):
                sl16 = pl.ds(j * D, D)
                v = idx_v[sl16]
                low = jnp.bitwise_and(v, 32767)
                p = (jnp.bitwise_and(v, ~32767)
                     | jnp.left_shift(jnp.bitwise_and(low, 4095), 3)
                     | jnp.right_shift(low, 12))
                idxp_v[sl16] = p
            cps = []
            for s in range(CHUNK):
                sl = pl.ds(s * NSLOT, NSLOT)
                cps.append(pltpu.async_copy(
                    emb_hbm.at[idxp_v.at[sl]], rows_v.at[sl], sem_e))
                cps.append(pltpu.async_copy(
                    fc_hbm.at[idx_v.at[sl]], fc_v.at[sl], sem_f))
            for cp in cps:
                cp.wait()

            def s_body(s, carry2):
                rb = s * NSLOT
                iv = [idx_v[pl.ds(rb + j * D, D)] for j in range(NV)]
                nz = [v != PAD for v in iv]

                def pcnt(j, lo=None, hi=None):
                    m = nz[j]
                    if lo is not None:
                        m = jnp.logical_and(m, lane >= lo)
                    if hi is not None:
                        m = jnp.logical_and(m, lane < hi)
                    return plsc.all_reduce_population_count(m)

                cg = pcnt(0, lo=3) + pcnt(1, hi=7)
                cw = pcnt(1, lo=7) + pcnt(2) + pcnt(3) + pcnt(4, hi=9)
                cd = pcnt(4, lo=9) + pcnt(5)
                inv_g = 1.0 / (cg.astype(jnp.float32) + 1e-08)
                inv_w = 1.0 / (cw.astype(jnp.float32) + 1e-08)
                inv_d = 1.0 / (cd.astype(jnp.float32) + 1e-08)

                invs = (None, None, None, inv_g, inv_w, inv_d)
                for f, (lo, hi) in enumerate(SEGS):
                    acc = rows_v[rb + lo]
                    for r in range(lo + 1, hi):
                        acc = acc + rows_v[rb + r]
                    if invs[f] is not None:
                        acc = acc * invs[f]
                    out_v[s, pl.ds(f * D, D)] = acc

                one = jnp.ones((D,), jnp.float32)
                scales = (
                    jnp.where(lane < 3, one, inv_g),
                    jnp.where(lane < 7, inv_g, inv_w),
                    inv_w,
                    inv_w,
                    jnp.where(lane < 9, inv_w, inv_d),
                    inv_d,
                )
                part = jnp.zeros((D,), jnp.float32)
                for j in range(NV):
                    part = part + fc_v[pl.ds(rb + j * D, D)] * scales[j]
                fcp_v[s] = part
                return carry2

            lax.fori_loop(0, CHUNK, s_body, 0)
            pltpu.sync_copy(out_v, embx_hbm.at[pl.ds(s0, CHUNK), :])
            pltpu.sync_copy(fcp_v, fcp_hbm.at[pl.ds(s0, CHUNK), :])
            return carry

        lax.fori_loop(0, NCHUNK, chunk_body, 0)

    return k(emb_table, fc_flat, idx_flat)


def _tc_body(embx_ref, fcp_ref, W1_ref, b1_ref, W2_ref, b2_ref,
             W3_ref, b3_ref, bias_ref, out_ref):
    ex = embx_ref[...]
    fm1 = bias_ref[0, 0] + jnp.sum(fcp_ref[...], axis=1, keepdims=True)

    s1 = jnp.sum(ex, axis=1, keepdims=True)
    s2 = jnp.sum(ex * ex, axis=1, keepdims=True)
    fm2 = 0.5 * (s1 * s1 - s2)

    h = jnp.maximum(
        jnp.dot(ex, W1_ref[...], preferred_element_type=jnp.float32)
        + b1_ref[...], 0.0)
    h = jnp.maximum(
        jnp.dot(h, W2_ref[...], preferred_element_type=jnp.float32)
        + b2_ref[...], 0.0)
    mlp = (jnp.dot(h, W3_ref[...], preferred_element_type=jnp.float32)
           + b3_ref[0, 0])

    out_ref[...] = jax.nn.sigmoid(fm1 + fm2 + mlp)


def kernel(user, item, genres, writers, directors, year,
           emb_table, fc_table, bias, W1, b1, W2, b2, W3, b3):
    i32 = jnp.int32
    idx_all = jnp.concatenate([
        user[:, None].astype(i32), item[:, None].astype(i32),
        year[:, None].astype(i32), genres.astype(i32),
        writers.astype(i32), directors.astype(i32),
        jnp.full((B, 3), PAD, dtype=i32),
    ], axis=1)

    emb_lin = _format_table(emb_table.T)
    embx, fcp = _sc_gather_pool(
        emb_lin, fc_table[:, 0], idx_all.reshape(-1))

    W1p = jnp.concatenate([W1[0:32], W1[80:96], W1[32:80]], axis=0)

    bm = 2048
    grid = (B // bm,)
    full = lambda i: (0, 0)
    y = pl.pallas_call(
        _tc_body,
        grid=grid,
        in_specs=[
            pl.BlockSpec((bm, NSLOT), lambda i: (i, 0)),
            pl.BlockSpec((bm, D), lambda i: (i, 0)),
            pl.BlockSpec((MLP_IN, H1), full),
            pl.BlockSpec((1, H1), full),
            pl.BlockSpec((H1, H2), full),
            pl.BlockSpec((1, H2), full),
            pl.BlockSpec((H2, 1), full),
            pl.BlockSpec((1, 1), full),
            pl.BlockSpec((1, 1), full),
        ],
        out_specs=pl.BlockSpec((bm, 1), lambda i: (i, 0)),
        out_shape=jax.ShapeDtypeStruct((B, 1), jnp.float32),
    )(embx, fcp, W1p, b1.reshape(1, H1), W2, b2.reshape(1, H2),
      W3, b3.reshape(1, 1), bias.reshape(1, 1))
    return y[:, 0]


# speedup vs baseline: 1.1432x; 1.0024x over previous
import functools

import jax
import jax.numpy as jnp
from jax import lax
from jax.experimental import pallas as pl
from jax.experimental.pallas import tpu as pltpu
from jax.experimental.pallas import tpu_sc as plsc

V = 900200
PAD = 600000
D = 16
B = 16384
NSLOT = 96
MLP_IN = 96
H1, H2 = 256, 128

SEGS = ((0, 1), (1, 2), (2, 3), (3, 23), (23, 73), (73, 93))

NC = 2
NS = 16
NW = NC * NS
PER_W = B // NW
CHUNK = 8
NCHUNK = PER_W // CHUNK
CIDX = CHUNK * NSLOT
NV = NSLOT // D


TBV = 32768
NTB = 28
VP = TBV * NTB


def _tc_transpose_body(in_ref, out_ref):
    x = in_ref[...]
    def _mask_tail(xv):
        colg = (TBV * (NTB - 1)
                + lax.broadcasted_iota(jnp.int32, (16, TBV), 1))
        return jnp.where(colg < V, xv, 0.0)

    x = lax.cond(pl.program_id(0) == NTB - 1, _mask_tail, lambda xv: xv, x)
    row = lax.broadcasted_iota(jnp.int32, (16, 128), 0)
    col = lax.broadcasted_iota(jnp.int32, (16, 128), 1)
    acc = jnp.zeros((TBV // 8, 128), jnp.float32)
    for a in range(8):
        place = (col == row + 16 * a).astype(jnp.float32)
        acc = acc + lax.dot_general(
            x[:, 4096 * a:4096 * (a + 1)], place,
            (((0,), (0,)), ((), ())),
            preferred_element_type=jnp.float32)
    out_ref[...] = acc


def _format_table(emb_t):
    out = pl.pallas_call(
        _tc_transpose_body,
        grid=(NTB,),
        in_specs=[pl.BlockSpec((16, TBV), lambda i: (0, i))],
        out_specs=pl.BlockSpec((TBV // 8, 128), lambda i: (i, 0)),
        out_shape=jax.ShapeDtypeStruct((VP // 8, 128), jnp.float32),
    )(emb_t)
    return out.reshape(VP, D)


def _sc_gather_pool(emb_table, fc_flat, idx_flat):
    mesh = plsc.VectorSubcoreMesh(core_axis_name="c", subcore_axis_name="s")

    @functools.partial(
        pl.kernel,
        mesh=mesh,
        compiler_params=pltpu.CompilerParams(use_tc_tiling_on_sc=False,
                                             needs_layout_passes=False),
        out_type=(
            jax.ShapeDtypeStruct((B, NSLOT), jnp.float32),
            jax.ShapeDtypeStruct((B, D), jnp.float32),
        ),
        scratch_types=[
            pltpu.VMEM((CIDX,), jnp.int32),
            pltpu.VMEM((CIDX,), jnp.int32),
            pltpu.VMEM((CIDX, D), jnp.float32),
            pltpu.VMEM((CIDX,), jnp.float32),
            pltpu.VMEM((CIDX,), jnp.int32),
            pltpu.VMEM((CIDX,), jnp.int32),
            pltpu.VMEM((CIDX, D), jnp.float32),
            pltpu.VMEM((CIDX,), jnp.float32),
            pltpu.VMEM((CHUNK, NSLOT), jnp.float32),
            pltpu.VMEM((CHUNK, D), jnp.float32),
            pltpu.SemaphoreType.DMA,
            pltpu.SemaphoreType.DMA,
            pltpu.SemaphoreType.DMA,
            pltpu.SemaphoreType.DMA,
        ],
    )
    def k(emb_hbm, fc_hbm, idx_hbm,
          embx_hbm, fcp_hbm,
          idx_v0, idxp_v0, rows_v0, fc_v0,
          idx_v1, idxp_v1, rows_v1, fc_v1,
          out_v, fcp_v, sem_e0, sem_f0, sem_e1, sem_f1):
        wid = lax.axis_index("s") * NC + lax.axis_index("c")
        lane = lax.broadcasted_iota(jnp.int32, (D,), 0)
        buf0 = (idx_v0, idxp_v0, rows_v0, fc_v0, sem_e0, sem_f0)
        buf1 = (idx_v1, idxp_v1, rows_v1, fc_v1, sem_e1, sem_f1)

        def issue(c, buf):
            idx_v, idxp_v, rows_v, fc_v, sem_e, sem_f = buf
            s0 = wid * PER_W + c * CHUNK
            pltpu.sync_copy(idx_hbm.at[pl.ds(s0 * NSLOT, CIDX)], idx_v)
            for j in range(CIDX // D):
                sl16 = pl.ds(j * D, D)
                v = idx_v[sl16]
                low = jnp.bitwise_and(v, 32767)
                p = (jnp.bitwise_and(v, ~32767)
                     | jnp.left_shift(jnp.bitwise_and(low, 4095), 3)
                     | jnp.right_shift(low, 12))
                idxp_v[sl16] = p
            for s in range(CHUNK):
                sl = pl.ds(s * NSLOT, NSLOT)
                pltpu.async_copy(
                    emb_hbm.at[idxp_v.at[sl]], rows_v.at[sl], sem_e)
                pltpu.async_copy(
                    fc_hbm.at[idx_v.at[sl]], fc_v.at[sl], sem_f)

        def drain(buf):
            idx_v, idxp_v, rows_v, fc_v, sem_e, sem_f = buf
            pltpu.make_async_copy(
                emb_hbm.at[pl.ds(0, CIDX)], rows_v, sem_e).wait()
            pltpu.make_async_copy(
                fc_hbm.at[pl.ds(0, CIDX)], fc_v, sem_f).wait()

        def compute(c, buf):
            idx_v, idxp_v, rows_v, fc_v, sem_e, sem_f = buf
            s0 = wid * PER_W + c * CHUNK

            def s_body(s, carry2):
                rb = s * NSLOT
                iv = [idx_v[pl.ds(rb + j * D, D)] for j in range(NV)]
                nz = [v != PAD for v in iv]

                def pcnt(j, lo=None, hi=None):
                    m = nz[j]
                    if lo is not None:
                        m = jnp.logical_and(m, lane >= lo)
                    if hi is not None:
                        m = jnp.logical_and(m, lane < hi)
                    return plsc.all_reduce_population_count(m)

                cg = pcnt(0, lo=3) + pcnt(1, hi=7)
                cw = pcnt(1, lo=7) + pcnt(2) + pcnt(3) + pcnt(4, hi=9)
                cd = pcnt(4, lo=9) + pcnt(5)
                inv_g = 1.0 / (cg.astype(jnp.float32) + 1e-08)
                inv_w = 1.0 / (cw.astype(jnp.float32) + 1e-08)
                inv_d = 1.0 / (cd.astype(jnp.float32) + 1e-08)

                invs = (None, None, None, inv_g, inv_w, inv_d)
                for f, (lo, hi) in enumerate(SEGS):
                    acc = rows_v[rb + lo]
                    for r in range(lo + 1, hi):
                        acc = acc + rows_v[rb + r]
                    if invs[f] is not None:
                        acc = acc * invs[f]
                    out_v[s, pl.ds(f * D, D)] = acc

                one = jnp.ones((D,), jnp.float32)
                scales = (
                    jnp.where(lane < 3, one, inv_g),
                    jnp.where(lane < 7, inv_g, inv_w),
                    inv_w,
                    inv_w,
                    jnp.where(lane < 9, inv_w, inv_d),
                    inv_d,
                )
                part = jnp.zeros((D,), jnp.float32)
                for j in range(NV):
                    part = part + fc_v[pl.ds(rb + j * D, D)] * scales[j]
                fcp_v[s] = part
                return carry2

            lax.fori_loop(0, CHUNK, s_body, 0)
            pltpu.sync_copy(out_v, embx_hbm.at[pl.ds(s0, CHUNK), :])
            pltpu.sync_copy(fcp_v, fcp_hbm.at[pl.ds(s0, CHUNK), :])

        issue(0, buf0)

        def pair(g, carry):
            c0 = 2 * g
            drain(buf0)
            issue(c0 + 1, buf1)
            compute(c0, buf0)
            drain(buf1)
            issue(c0 + 2, buf0)
            compute(c0 + 1, buf1)
            return carry

        lax.fori_loop(0, NCHUNK // 2 - 1, pair, 0)
        cl = NCHUNK - 2
        drain(buf0)
        issue(cl + 1, buf1)
        compute(cl, buf0)
        drain(buf1)
        compute(cl + 1, buf1)

    return k(emb_table, fc_flat, idx_flat)


def _tc_body(embx_ref, fcp_ref, W1_ref, b1_ref, W2_ref, b2_ref,
             W3_ref, b3_ref, bias_ref, out_ref):
    ex = embx_ref[...]
    fm1 = bias_ref[0, 0] + jnp.sum(fcp_ref[...], axis=1, keepdims=True)

    s1 = jnp.sum(ex, axis=1, keepdims=True)
    s2 = jnp.sum(ex * ex, axis=1, keepdims=True)
    fm2 = 0.5 * (s1 * s1 - s2)

    h = jnp.maximum(
        jnp.dot(ex, W1_ref[...], preferred_element_type=jnp.float32)
        + b1_ref[...], 0.0)
    h = jnp.maximum(
        jnp.dot(h, W2_ref[...], preferred_element_type=jnp.float32)
        + b2_ref[...], 0.0)
    mlp = (jnp.dot(h, W3_ref[...], preferred_element_type=jnp.float32)
           + b3_ref[0, 0])

    out_ref[...] = jax.nn.sigmoid(fm1 + fm2 + mlp)


def kernel(user, item, genres, writers, directors, year,
           emb_table, fc_table, bias, W1, b1, W2, b2, W3, b3):
    i32 = jnp.int32
    idx_all = jnp.concatenate([
        user[:, None].astype(i32), item[:, None].astype(i32),
        year[:, None].astype(i32), genres.astype(i32),
        writers.astype(i32), directors.astype(i32),
        jnp.full((B, 3), PAD, dtype=i32),
    ], axis=1)

    emb_lin = _format_table(emb_table.T)
    embx, fcp = _sc_gather_pool(
        emb_lin, fc_table[:, 0], idx_all.reshape(-1))

    W1p = jnp.concatenate([W1[0:32], W1[80:96], W1[32:80]], axis=0)

    bm = 2048
    grid = (B // bm,)
    full = lambda i: (0, 0)
    y = pl.pallas_call(
        _tc_body,
        grid=grid,
        in_specs=[
            pl.BlockSpec((bm, NSLOT), lambda i: (i, 0)),
            pl.BlockSpec((bm, D), lambda i: (i, 0)),
            pl.BlockSpec((MLP_IN, H1), full),
            pl.BlockSpec((1, H1), full),
            pl.BlockSpec((H1, H2), full),
            pl.BlockSpec((1, H2), full),
            pl.BlockSpec((H2, 1), full),
            pl.BlockSpec((1, 1), full),
            pl.BlockSpec((1, 1), full),
        ],
        out_specs=pl.BlockSpec((bm, 1), lambda i: (i, 0)),
        out_shape=jax.ShapeDtypeStruct((B, 1), jnp.float32),
    )(embx, fcp, W1p, b1.reshape(1, H1), W2, b2.reshape(1, H2),
      W3, b3.reshape(1, 1), bias.reshape(1, 1))
    return y[:, 0]
